# Initial kernel scaffold; baseline (speedup 1.0000x reference)
import jax
import jax.numpy as jnp
from jax.experimental import pallas as pl


def kernel(x, p, edge_attr, edge_index, batch, embed_W, embed_b, edge_W, edge_b, msg_W, msg_b, hup_W, hup_b, eup_W, eup_b):
    raise NotImplementedError("write your pallas kernel here")


# speedup vs baseline: 6.3223x; 6.3223x over previous
import functools

import jax
import jax.numpy as jnp
from jax import lax
from jax.experimental import pallas as pl
from jax.experimental.pallas import tpu as pltpu
from jax.experimental.pallas import tpu_sc as plsc

_N = 10000
_E = 320000
_G = 64
_H = 64
_DX = 128
_DP = 16
_DE = 16

_NC = 2
_NS = 16
_NW = _NC * _NS
_CHUNK = 128
_NP = 10112
_RPT = _NP // _NS

_NCHUNKS = _E // _CHUNK
_Q, _R = divmod(_NCHUNKS, _NW)

_mesh = plsc.VectorSubcoreMesh(core_axis_name="c", subcore_axis_name="s")


def _sc_scatter_body(gather_width, rows_hbm, send_hbm, rec_hbm, zero_hbm,
                     out_hbm, acc_sp, stage_v, idx_g, idx_s, rows_v):
    c = lax.axis_index("c")
    s = lax.axis_index("s")
    wid = c * _NS + s
    r0 = s * _RPT

    pltpu.sync_copy(zero_hbm.at[pl.ds(r0, _RPT)], stage_v)
    pltpu.sync_copy(stage_v, acc_sp.at[pl.ds(r0, _RPT)])
    plsc.subcore_barrier()

    nch = _Q + jnp.where(wid < _R, 1, 0)
    ch0 = wid * _Q + jnp.minimum(wid, _R)

    def body(j, carry):
        base = (ch0 + j) * _CHUNK
        pltpu.sync_copy(rec_hbm.at[pl.ds(base, _CHUNK)], idx_s)
        if gather_width:
            pltpu.sync_copy(send_hbm.at[pl.ds(base, _CHUNK)], idx_g)
            pltpu.sync_copy(rows_hbm.at[idx_g], rows_v)
        else:
            pltpu.sync_copy(rows_hbm.at[pl.ds(base, _CHUNK)], rows_v)
        pltpu.sync_copy(rows_v, acc_sp.at[idx_s], add=True)
        return carry

    lax.fori_loop(0, nch, body, 0)
    plsc.subcore_barrier()
    pltpu.sync_copy(acc_sp.at[pl.ds(r0, _RPT)], stage_v)
    pltpu.sync_copy(stage_v, out_hbm.at[c, pl.ds(r0, _RPT)])


def _make_gs_call(width):
    body = functools.partial(_sc_scatter_body, width)
    return pl.kernel(
        body,
        out_type=jax.ShapeDtypeStruct((_NC, _NP, width), jnp.float32),
        mesh=_mesh,
        scratch_types=[
            pltpu.VMEM_SHARED((_NP, width), jnp.float32),
            pltpu.VMEM((_RPT, width), jnp.float32),
            pltpu.VMEM((_CHUNK,), jnp.int32),
            pltpu.VMEM((_CHUNK,), jnp.int32),
            pltpu.VMEM((_CHUNK, width), jnp.float32),
        ],
        compiler_params=pltpu.CompilerParams(use_tc_tiling_on_sc=False),
        name=f"sc_gs_{width}",
    )


def _make_ea_call(width):
    def body(rows_hbm, rec_hbm, zero_hbm, out_hbm, acc_sp, stage_v,
             idx_s, rows_v):
        _sc_scatter_body(0, rows_hbm, None, rec_hbm, zero_hbm, out_hbm,
                         acc_sp, stage_v, None, idx_s, rows_v)

    return pl.kernel(
        body,
        out_type=jax.ShapeDtypeStruct((_NC, _NP, width), jnp.float32),
        mesh=_mesh,
        scratch_types=[
            pltpu.VMEM_SHARED((_NP, width), jnp.float32),
            pltpu.VMEM((_RPT, width), jnp.float32),
            pltpu.VMEM((_CHUNK,), jnp.int32),
            pltpu.VMEM((_CHUNK, width), jnp.float32),
        ],
        compiler_params=pltpu.CompilerParams(use_tc_tiling_on_sc=False),
        name=f"sc_ea_{width}",
    )


_gs_call = _make_gs_call(_H)
_ea_call = _make_ea_call(32)


def _dot(a, b):
    return jnp.dot(a, b, preferred_element_type=jnp.float32,
                   precision=lax.Precision.HIGHEST)


def _embed_body(x_ref, p_ref, w_ref, b_ref, o_ref):
    w = w_ref[...]
    o_ref[...] = (_dot(x_ref[...], w[0:_DX]) + _dot(p_ref[...], w[_DX:])
                  + b_ref[...])


def _se0_body(ead_ref, ew_ref, eb_ref, se_ref, deg_ref):
    ead = ead_ref[0] + ead_ref[1]
    deg = ead[:, _DE:_DE + 1]
    se_ref[...] = _dot(ead[:, 0:_DE], ew_ref[...]) + deg * eb_ref[...]
    deg_ref[...] = deg


def _layer_core(h_ref, se_ref, deg_ref, gp_ref, mw_ref, hw_ref, mb_ref,
                hb_ref):
    g = gp_ref[0] + gp_ref[1]
    h = h_ref[...]
    deg = deg_ref[...]
    dh = deg * h
    mw = mw_ref[...]
    agg = (_dot(g, mw[0:_H]) + _dot(dh, mw[_H:2 * _H])
           + _dot(se_ref[...], mw[2 * _H:]) + deg * mb_ref[...])
    hw = hw_ref[...]
    hn = _dot(h, hw[0:_H]) + _dot(agg, hw[_H:]) + hb_ref[...]
    return hn, g, dh


def _layer_body(h_ref, se_ref, deg_ref, gp_ref, mw_ref, hw_ref, ew_ref,
                mb_ref, hb_ref, eb_ref, hn_ref, sn_ref):
    hn, g, dh = _layer_core(h_ref, se_ref, deg_ref, gp_ref, mw_ref, hw_ref,
                            mb_ref, hb_ref)
    ew = ew_ref[...]
    sn_ref[...] = (_dot(g, ew[0:_H]) + _dot(dh, ew[_H:2 * _H])
                   + _dot(se_ref[...], ew[2 * _H:]) + deg_ref[...] * eb_ref[...])
    hn_ref[...] = hn


def _final_body(h_ref, se_ref, deg_ref, gp_ref, batch_ref, mw_ref, hw_ref,
                mb_ref, hb_ref, o_ref):
    hn, _, _ = _layer_core(h_ref, se_ref, deg_ref, gp_ref, mw_ref, hw_ref,
                           mb_ref, hb_ref)
    ids = lax.broadcasted_iota(jnp.int32, (_G, _RB), 0)
    onehot = (ids == batch_ref[0]).astype(jnp.float32)

    @pl.when(pl.program_id(0) == 0)
    def _():
        o_ref[...] = jnp.zeros_like(o_ref)

    o_ref[...] += _dot(onehot, hn)


_f32 = jnp.float32
_RB = 1264
_NB = _NP // _RB


def _rows(width):
    return pl.BlockSpec((_RB, width), lambda i: (i, 0))


def _full(*shape):
    return pl.BlockSpec(shape, lambda i: (0,) * len(shape))


def _part(width):
    return pl.BlockSpec((_NC, _RB, width), lambda i: (0, i, 0))


def _tc(body, in_specs, out_specs, out_shapes):
    return pl.pallas_call(body, grid=(_NB,), in_specs=in_specs,
                          out_specs=out_specs, out_shape=out_shapes)


def kernel(x, p, edge_attr, edge_index, batch, embed_W, embed_b, edge_W,
           edge_b, msg_W, msg_b, hup_W, hup_b, eup_W, eup_b):
    send = edge_index[0]
    rec = edge_index[1]
    pad_n = _NP - _N
    ea_pad = jnp.concatenate(
        [edge_attr, jnp.ones((_E, 1), _f32), jnp.zeros((_E, 32 - _DE - 1), _f32)],
        axis=1)
    z32 = jnp.zeros((_NP, 32), _f32)
    z64 = jnp.zeros((_NP, _H), _f32)
    xp = jnp.pad(x, ((0, pad_n), (0, 0)))
    pp = jnp.pad(p, ((0, pad_n), (0, 0)))
    batch3d = jnp.pad(batch, (0, pad_n), constant_values=-1).reshape(
        _NB, 1, _RB)

    ead_part = _ea_call(ea_pad, rec, z32)

    h = _tc(_embed_body,
            [_rows(_DX), _rows(_DP), _full(_DX + _DP, _H), _full(1, _H)],
            _rows(_H), jax.ShapeDtypeStruct((_NP, _H), _f32))(
        xp, pp, embed_W, embed_b.reshape(1, _H))

    se, deg = _tc(_se0_body,
                  [_part(32), _full(_DE, _H), _full(1, _H)],
                  (_rows(_H), _rows(1)),
                  (jax.ShapeDtypeStruct((_NP, _H), _f32),
                   jax.ShapeDtypeStruct((_NP, 1), _f32)))(
        ead_part, edge_W, edge_b.reshape(1, _H))

    layer_specs = [_rows(_H), _rows(_H), _rows(1), _part(_H),
                   _full(3 * _H, _H), _full(2 * _H, _H), _full(3 * _H, _H),
                   _full(1, _H), _full(1, _H), _full(1, _H)]
    for l in range(2):
        gp = _gs_call(h, send, rec, z64)
        h, se = _tc(_layer_body, layer_specs, (_rows(_H), _rows(_H)),
                    (jax.ShapeDtypeStruct((_NP, _H), _f32),
                     jax.ShapeDtypeStruct((_NP, _H), _f32)))(
            h, se, deg, gp, msg_W[l], hup_W[l], eup_W[l],
            msg_b[l].reshape(1, _H), hup_b[l].reshape(1, _H),
            eup_b[l].reshape(1, _H))

    gp = _gs_call(h, send, rec, z64)
    final_specs = [_rows(_H), _rows(_H), _rows(1), _part(_H),
                   pl.BlockSpec((1, 1, _RB), lambda i: (i, 0, 0)),
                   _full(3 * _H, _H), _full(2 * _H, _H),
                   _full(1, _H), _full(1, _H)]
    out = _tc(_final_body, final_specs, _full(_G, _H),
              jax.ShapeDtypeStruct((_G, _H), _f32))(
        h, se, deg, gp, batch3d, msg_W[2], hup_W[2],
        msg_b[2].reshape(1, _H), hup_b[2].reshape(1, _H))
    return out


# speedup vs baseline: 9.6106x; 1.5201x over previous
import functools

import jax
import jax.numpy as jnp
from jax import lax
from jax.experimental import pallas as pl
from jax.experimental.pallas import tpu as pltpu
from jax.experimental.pallas import tpu_sc as plsc

_N = 10000
_E = 320000
_G = 64
_H = 64
_DX = 128
_DP = 16
_DE = 16

_NC = 2
_NS = 16
_NW = _NC * _NS
_CHUNK = 512
_NP = 10112
_RPT = _NP // _NS

_NCHUNKS = _E // _CHUNK
_Q, _R = divmod(_NCHUNKS, _NW)

_mesh = plsc.VectorSubcoreMesh(core_axis_name="c", subcore_axis_name="s")


def _sc_scatter_body(gather_width, rows_hbm, send_hbm, rec_hbm, zero_hbm,
                     out_hbm, acc_sp, stage_v, idx_g, idx_s, rows_v):
    c = lax.axis_index("c")
    s = lax.axis_index("s")
    wid = c * _NS + s
    r0 = s * _RPT

    pltpu.sync_copy(zero_hbm.at[pl.ds(r0, _RPT)], stage_v)
    pltpu.sync_copy(stage_v, acc_sp.at[pl.ds(r0, _RPT)])
    plsc.subcore_barrier()

    nch = _Q + jnp.where(wid < _R, 1, 0)
    ch0 = wid * _Q + jnp.minimum(wid, _R)

    def body(j, carry):
        base = (ch0 + j) * _CHUNK
        pltpu.sync_copy(rec_hbm.at[pl.ds(base, _CHUNK)], idx_s)
        if gather_width:
            pltpu.sync_copy(send_hbm.at[pl.ds(base, _CHUNK)], idx_g)
            pltpu.sync_copy(rows_hbm.at[idx_g], rows_v)
        else:
            pltpu.sync_copy(rows_hbm.at[pl.ds(base, _CHUNK)], rows_v)
        pltpu.sync_copy(rows_v, acc_sp.at[idx_s], add=True)
        return carry

    lax.fori_loop(0, nch, body, 0)
    plsc.subcore_barrier()
    pltpu.sync_copy(acc_sp.at[pl.ds(r0, _RPT)], stage_v)
    pltpu.sync_copy(stage_v, out_hbm.at[c, pl.ds(r0, _RPT)])


def _make_gs_call(width):
    body = functools.partial(_sc_scatter_body, width)
    return pl.kernel(
        body,
        out_type=jax.ShapeDtypeStruct((_NC, _NP, width), jnp.float32),
        mesh=_mesh,
        scratch_types=[
            pltpu.VMEM_SHARED((_NP, width), jnp.float32),
            pltpu.VMEM((_RPT, width), jnp.float32),
            pltpu.VMEM((_CHUNK,), jnp.int32),
            pltpu.VMEM((_CHUNK,), jnp.int32),
            pltpu.VMEM((_CHUNK, width), jnp.float32),
        ],
        compiler_params=pltpu.CompilerParams(use_tc_tiling_on_sc=False),
        name=f"sc_gs_{width}",
    )


def _make_ea_call(width):
    def body(rows_hbm, rec_hbm, zero_hbm, out_hbm, acc_sp, stage_v,
             idx_s, rows_v):
        _sc_scatter_body(0, rows_hbm, None, rec_hbm, zero_hbm, out_hbm,
                         acc_sp, stage_v, None, idx_s, rows_v)

    return pl.kernel(
        body,
        out_type=jax.ShapeDtypeStruct((_NC, _NP, width), jnp.float32),
        mesh=_mesh,
        scratch_types=[
            pltpu.VMEM_SHARED((_NP, width), jnp.float32),
            pltpu.VMEM((_RPT, width), jnp.float32),
            pltpu.VMEM((_CHUNK,), jnp.int32),
            pltpu.VMEM((_CHUNK, width), jnp.float32),
        ],
        compiler_params=pltpu.CompilerParams(use_tc_tiling_on_sc=False),
        name=f"sc_ea_{width}",
    )


_gs_call = _make_gs_call(_H)
_ea_call = _make_ea_call(32)


def _dot(a, b):
    return jnp.dot(a, b, preferred_element_type=jnp.float32,
                   precision=lax.Precision.HIGHEST)


def _embed_body(x_ref, p_ref, w_ref, b_ref, o_ref):
    w = w_ref[...]
    o_ref[...] = (_dot(x_ref[...], w[0:_DX]) + _dot(p_ref[...], w[_DX:])
                  + b_ref[...])


def _se0_body(ead_ref, ew_ref, eb_ref, se_ref, deg_ref):
    ead = ead_ref[0] + ead_ref[1]
    deg = ead[:, _DE:_DE + 1]
    se_ref[...] = _dot(ead[:, 0:_DE], ew_ref[...]) + deg * eb_ref[...]
    deg_ref[...] = deg


def _layer_core(h_ref, se_ref, deg_ref, gp_ref, mw_ref, hw_ref, mb_ref,
                hb_ref):
    g = gp_ref[0] + gp_ref[1]
    h = h_ref[...]
    deg = deg_ref[...]
    dh = deg * h
    mw = mw_ref[...]
    agg = (_dot(g, mw[0:_H]) + _dot(dh, mw[_H:2 * _H])
           + _dot(se_ref[...], mw[2 * _H:]) + deg * mb_ref[...])
    hw = hw_ref[...]
    hn = _dot(h, hw[0:_H]) + _dot(agg, hw[_H:]) + hb_ref[...]
    return hn, g, dh


def _layer_body(h_ref, se_ref, deg_ref, gp_ref, mw_ref, hw_ref, ew_ref,
                mb_ref, hb_ref, eb_ref, hn_ref, sn_ref):
    hn, g, dh = _layer_core(h_ref, se_ref, deg_ref, gp_ref, mw_ref, hw_ref,
                            mb_ref, hb_ref)
    ew = ew_ref[...]
    sn_ref[...] = (_dot(g, ew[0:_H]) + _dot(dh, ew[_H:2 * _H])
                   + _dot(se_ref[...], ew[2 * _H:]) + deg_ref[...] * eb_ref[...])
    hn_ref[...] = hn


def _final_body(h_ref, se_ref, deg_ref, gp_ref, batch_ref, mw_ref, hw_ref,
                mb_ref, hb_ref, o_ref):
    hn, _, _ = _layer_core(h_ref, se_ref, deg_ref, gp_ref, mw_ref, hw_ref,
                           mb_ref, hb_ref)
    ids = lax.broadcasted_iota(jnp.int32, (_G, _RB), 0)
    onehot = (ids == batch_ref[0]).astype(jnp.float32)

    @pl.when(pl.program_id(0) == 0)
    def _():
        o_ref[...] = jnp.zeros_like(o_ref)

    o_ref[...] += _dot(onehot, hn)


_f32 = jnp.float32
_RB = 1264
_NB = _NP // _RB


def _rows(width):
    return pl.BlockSpec((_RB, width), lambda i: (i, 0))


def _full(*shape):
    return pl.BlockSpec(shape, lambda i: (0,) * len(shape))


def _part(width):
    return pl.BlockSpec((_NC, _RB, width), lambda i: (0, i, 0))


def _tc(body, in_specs, out_specs, out_shapes):
    return pl.pallas_call(body, grid=(_NB,), in_specs=in_specs,
                          out_specs=out_specs, out_shape=out_shapes)


def kernel(x, p, edge_attr, edge_index, batch, embed_W, embed_b, edge_W,
           edge_b, msg_W, msg_b, hup_W, hup_b, eup_W, eup_b):
    send = edge_index[0]
    rec = edge_index[1]
    pad_n = _NP - _N
    ea_pad = jnp.concatenate(
        [edge_attr, jnp.ones((_E, 1), _f32), jnp.zeros((_E, 32 - _DE - 1), _f32)],
        axis=1)
    z32 = jnp.zeros((_NP, 32), _f32)
    z64 = jnp.zeros((_NP, _H), _f32)
    xp = jnp.pad(x, ((0, pad_n), (0, 0)))
    pp = jnp.pad(p, ((0, pad_n), (0, 0)))
    batch3d = jnp.pad(batch, (0, pad_n), constant_values=-1).reshape(
        _NB, 1, _RB)

    ead_part = _ea_call(ea_pad, rec, z32)

    h = _tc(_embed_body,
            [_rows(_DX), _rows(_DP), _full(_DX + _DP, _H), _full(1, _H)],
            _rows(_H), jax.ShapeDtypeStruct((_NP, _H), _f32))(
        xp, pp, embed_W, embed_b.reshape(1, _H))

    se, deg = _tc(_se0_body,
                  [_part(32), _full(_DE, _H), _full(1, _H)],
                  (_rows(_H), _rows(1)),
                  (jax.ShapeDtypeStruct((_NP, _H), _f32),
                   jax.ShapeDtypeStruct((_NP, 1), _f32)))(
        ead_part, edge_W, edge_b.reshape(1, _H))

    layer_specs = [_rows(_H), _rows(_H), _rows(1), _part(_H),
                   _full(3 * _H, _H), _full(2 * _H, _H), _full(3 * _H, _H),
                   _full(1, _H), _full(1, _H), _full(1, _H)]
    for l in range(2):
        gp = _gs_call(h, send, rec, z64)
        h, se = _tc(_layer_body, layer_specs, (_rows(_H), _rows(_H)),
                    (jax.ShapeDtypeStruct((_NP, _H), _f32),
                     jax.ShapeDtypeStruct((_NP, _H), _f32)))(
            h, se, deg, gp, msg_W[l], hup_W[l], eup_W[l],
            msg_b[l].reshape(1, _H), hup_b[l].reshape(1, _H),
            eup_b[l].reshape(1, _H))

    gp = _gs_call(h, send, rec, z64)
    final_specs = [_rows(_H), _rows(_H), _rows(1), _part(_H),
                   pl.BlockSpec((1, 1, _RB), lambda i: (i, 0, 0)),
                   _full(3 * _H, _H), _full(2 * _H, _H),
                   _full(1, _H), _full(1, _H)]
    out = _tc(_final_body, final_specs, _full(_G, _H),
              jax.ShapeDtypeStruct((_G, _H), _f32))(
        h, se, deg, gp, batch3d, msg_W[2], hup_W[2],
        msg_b[2].reshape(1, _H), hup_b[2].reshape(1, _H))
    return out


# speedup vs baseline: 10.1792x; 1.0592x over previous
import functools

import jax
import jax.numpy as jnp
from jax import lax
from jax.experimental import pallas as pl
from jax.experimental.pallas import tpu as pltpu
from jax.experimental.pallas import tpu_sc as plsc

_N = 10000
_E = 320000
_G = 64
_H = 64
_DX = 128
_DP = 16
_DE = 16

_NC = 2
_NS = 16
_NW = _NC * _NS
_CHUNK = 1280
_NP = 10112
_RPT = _NP // _NS

_NCHUNKS = _E // _CHUNK
_Q, _R = divmod(_NCHUNKS, _NW)

_mesh = plsc.VectorSubcoreMesh(core_axis_name="c", subcore_axis_name="s")


def _sc_scatter_body(gather_width, rows_hbm, send_hbm, rec_hbm, zero_hbm,
                     out_hbm, acc_sp, idx_g, idx_s, rows_v):
    c = lax.axis_index("c")
    s = lax.axis_index("s")
    wid = c * _NS + s
    r0 = s * _RPT

    pltpu.sync_copy(zero_hbm.at[pl.ds(r0, _RPT)], rows_v.at[pl.ds(0, _RPT)])
    pltpu.sync_copy(rows_v.at[pl.ds(0, _RPT)], acc_sp.at[pl.ds(r0, _RPT)])
    plsc.subcore_barrier()

    nch = _Q + jnp.where(wid < _R, 1, 0)
    ch0 = wid * _Q + jnp.minimum(wid, _R)

    def body(j, carry):
        base = (ch0 + j) * _CHUNK
        pltpu.sync_copy(rec_hbm.at[pl.ds(base, _CHUNK)], idx_s)
        if gather_width:
            pltpu.sync_copy(send_hbm.at[pl.ds(base, _CHUNK)], idx_g)
            pltpu.sync_copy(rows_hbm.at[idx_g], rows_v)
        else:
            pltpu.sync_copy(rows_hbm.at[pl.ds(base, _CHUNK)], rows_v)
        pltpu.sync_copy(rows_v, acc_sp.at[idx_s], add=True)
        return carry

    lax.fori_loop(0, nch, body, 0)
    plsc.subcore_barrier()
    pltpu.sync_copy(acc_sp.at[pl.ds(r0, _RPT)], rows_v.at[pl.ds(0, _RPT)])
    pltpu.sync_copy(rows_v.at[pl.ds(0, _RPT)], out_hbm.at[c, pl.ds(r0, _RPT)])


def _make_gs_call(width):
    body = functools.partial(_sc_scatter_body, width)
    return pl.kernel(
        body,
        out_type=jax.ShapeDtypeStruct((_NC, _NP, width), jnp.float32),
        mesh=_mesh,
        scratch_types=[
            pltpu.VMEM_SHARED((_NP, width), jnp.float32),
            pltpu.VMEM((_CHUNK,), jnp.int32),
            pltpu.VMEM((_CHUNK,), jnp.int32),
            pltpu.VMEM((_CHUNK, width), jnp.float32),
        ],
        compiler_params=pltpu.CompilerParams(use_tc_tiling_on_sc=False),
        name=f"sc_gs_{width}",
    )


def _make_ea_call(width):
    def body(rows_hbm, rec_hbm, zero_hbm, out_hbm, acc_sp, idx_s, rows_v):
        _sc_scatter_body(0, rows_hbm, None, rec_hbm, zero_hbm, out_hbm,
                         acc_sp, None, idx_s, rows_v)

    return pl.kernel(
        body,
        out_type=jax.ShapeDtypeStruct((_NC, _NP, width), jnp.float32),
        mesh=_mesh,
        scratch_types=[
            pltpu.VMEM_SHARED((_NP, width), jnp.float32),
            pltpu.VMEM((_CHUNK,), jnp.int32),
            pltpu.VMEM((_CHUNK, width), jnp.float32),
        ],
        compiler_params=pltpu.CompilerParams(use_tc_tiling_on_sc=False),
        name=f"sc_ea_{width}",
    )


_gs_call = _make_gs_call(_H)
_ea_call = _make_ea_call(32)


def _dot(a, b):
    return jnp.dot(a, b, preferred_element_type=jnp.float32,
                   precision=lax.Precision.HIGHEST)


def _embed_body(x_ref, p_ref, w_ref, b_ref, o_ref):
    w = w_ref[...]
    o_ref[...] = (_dot(x_ref[...], w[0:_DX]) + _dot(p_ref[...], w[_DX:])
                  + b_ref[...])


def _se0_body(ead_ref, ew_ref, eb_ref, se_ref, deg_ref):
    ead = ead_ref[0] + ead_ref[1]
    deg = ead[:, _DE:_DE + 1]
    se_ref[...] = _dot(ead[:, 0:_DE], ew_ref[...]) + deg * eb_ref[...]
    deg_ref[...] = deg


def _layer_core(h_ref, se_ref, deg_ref, gp_ref, mw_ref, hw_ref, mb_ref,
                hb_ref):
    g = gp_ref[0] + gp_ref[1]
    h = h_ref[...]
    deg = deg_ref[...]
    dh = deg * h
    mw = mw_ref[...]
    agg = (_dot(g, mw[0:_H]) + _dot(dh, mw[_H:2 * _H])
           + _dot(se_ref[...], mw[2 * _H:]) + deg * mb_ref[...])
    hw = hw_ref[...]
    hn = _dot(h, hw[0:_H]) + _dot(agg, hw[_H:]) + hb_ref[...]
    return hn, g, dh


def _layer_body(h_ref, se_ref, deg_ref, gp_ref, mw_ref, hw_ref, ew_ref,
                mb_ref, hb_ref, eb_ref, hn_ref, sn_ref):
    hn, g, dh = _layer_core(h_ref, se_ref, deg_ref, gp_ref, mw_ref, hw_ref,
                            mb_ref, hb_ref)
    ew = ew_ref[...]
    sn_ref[...] = (_dot(g, ew[0:_H]) + _dot(dh, ew[_H:2 * _H])
                   + _dot(se_ref[...], ew[2 * _H:]) + deg_ref[...] * eb_ref[...])
    hn_ref[...] = hn


def _final_body(h_ref, se_ref, deg_ref, gp_ref, batch_ref, mw_ref, hw_ref,
                mb_ref, hb_ref, o_ref):
    hn, _, _ = _layer_core(h_ref, se_ref, deg_ref, gp_ref, mw_ref, hw_ref,
                           mb_ref, hb_ref)
    ids = lax.broadcasted_iota(jnp.int32, (_G, _RB), 0)
    onehot = (ids == batch_ref[0]).astype(jnp.float32)

    @pl.when(pl.program_id(0) == 0)
    def _():
        o_ref[...] = jnp.zeros_like(o_ref)

    o_ref[...] += _dot(onehot, hn)


_f32 = jnp.float32
_RB = 1264
_NB = _NP // _RB


def _rows(width):
    return pl.BlockSpec((_RB, width), lambda i: (i, 0))


def _full(*shape):
    return pl.BlockSpec(shape, lambda i: (0,) * len(shape))


def _part(width):
    return pl.BlockSpec((_NC, _RB, width), lambda i: (0, i, 0))


def _tc(body, in_specs, out_specs, out_shapes):
    return pl.pallas_call(body, grid=(_NB,), in_specs=in_specs,
                          out_specs=out_specs, out_shape=out_shapes)


def kernel(x, p, edge_attr, edge_index, batch, embed_W, embed_b, edge_W,
           edge_b, msg_W, msg_b, hup_W, hup_b, eup_W, eup_b):
    send = edge_index[0]
    rec = edge_index[1]
    pad_n = _NP - _N
    ea_pad = jnp.concatenate(
        [edge_attr, jnp.ones((_E, 1), _f32), jnp.zeros((_E, 32 - _DE - 1), _f32)],
        axis=1)
    z32 = jnp.zeros((_NP, 32), _f32)
    z64 = jnp.zeros((_NP, _H), _f32)
    xp = jnp.pad(x, ((0, pad_n), (0, 0)))
    pp = jnp.pad(p, ((0, pad_n), (0, 0)))
    batch3d = jnp.pad(batch, (0, pad_n), constant_values=-1).reshape(
        _NB, 1, _RB)

    ead_part = _ea_call(ea_pad, rec, z32)

    h = _tc(_embed_body,
            [_rows(_DX), _rows(_DP), _full(_DX + _DP, _H), _full(1, _H)],
            _rows(_H), jax.ShapeDtypeStruct((_NP, _H), _f32))(
        xp, pp, embed_W, embed_b.reshape(1, _H))

    se, deg = _tc(_se0_body,
                  [_part(32), _full(_DE, _H), _full(1, _H)],
                  (_rows(_H), _rows(1)),
                  (jax.ShapeDtypeStruct((_NP, _H), _f32),
                   jax.ShapeDtypeStruct((_NP, 1), _f32)))(
        ead_part, edge_W, edge_b.reshape(1, _H))

    layer_specs = [_rows(_H), _rows(_H), _rows(1), _part(_H),
                   _full(3 * _H, _H), _full(2 * _H, _H), _full(3 * _H, _H),
                   _full(1, _H), _full(1, _H), _full(1, _H)]
    for l in range(2):
        gp = _gs_call(h, send, rec, z64)
        h, se = _tc(_layer_body, layer_specs, (_rows(_H), _rows(_H)),
                    (jax.ShapeDtypeStruct((_NP, _H), _f32),
                     jax.ShapeDtypeStruct((_NP, _H), _f32)))(
            h, se, deg, gp, msg_W[l], hup_W[l], eup_W[l],
            msg_b[l].reshape(1, _H), hup_b[l].reshape(1, _H),
            eup_b[l].reshape(1, _H))

    gp = _gs_call(h, send, rec, z64)
    final_specs = [_rows(_H), _rows(_H), _rows(1), _part(_H),
                   pl.BlockSpec((1, 1, _RB), lambda i: (i, 0, 0)),
                   _full(3 * _H, _H), _full(2 * _H, _H),
                   _full(1, _H), _full(1, _H)]
    out = _tc(_final_body, final_specs, _full(_G, _H),
              jax.ShapeDtypeStruct((_G, _H), _f32))(
        h, se, deg, gp, batch3d, msg_W[2], hup_W[2],
        msg_b[2].reshape(1, _H), hup_b[2].reshape(1, _H))
    return out
